# Initial kernel scaffold; baseline (speedup 1.0000x reference)
import jax
import jax.numpy as jnp
from jax.experimental import pallas as pl


def kernel(x, edge_index, W1, b1, W2, b2, LW1, Lb1, LW2, Lb2):
    raise NotImplementedError("write your pallas kernel here")


# speedup vs baseline: 20.4519x; 20.4519x over previous
import functools

import jax
import jax.numpy as jnp
from jax import lax
from jax.experimental import pallas as pl
from jax.experimental.pallas import tpu as pltpu
from jax.experimental.pallas import tpu_sc as plsc

N = 10000
D = 128
E = 320000
H = 32

NC = 2
NS = 16
NW = NC * NS

NP = 10240
CH = 128
PT = 10112
NCHUNK = PT // CH
EP = NW * PT
ROWS_PER_TILE = NP // NS


def _sc_mesh():
    return plsc.VectorSubcoreMesh(core_axis_name="c", subcore_axis_name="s")


def _deg_body(dst_hbm, deg_hbm, idx_v, ones_v, zv, deg_sh):
    cid = lax.axis_index("c")
    sid = lax.axis_index("s")
    wid = cid * NS + sid

    for k in range(CH // 16):
        ones_v[pl.ds(16 * k, 16)] = jnp.full((16,), 1.0, jnp.float32)
        zv[pl.ds(16 * k, 16)] = jnp.zeros((16,), jnp.float32)
    for i in range(ROWS_PER_TILE // CH):
        pltpu.sync_copy(zv, deg_sh.at[pl.ds(sid * ROWS_PER_TILE + i * CH, CH)])
    plsc.subcore_barrier()

    def chunk(j, carry):
        base = wid * PT + j * CH
        pltpu.sync_copy(dst_hbm.at[pl.ds(base, CH)], idx_v)
        pltpu.sync_copy(ones_v, deg_sh.at[idx_v], add=True)
        return carry

    lax.fori_loop(0, NCHUNK, chunk, 0)
    plsc.subcore_barrier()
    pltpu.sync_copy(
        deg_sh.at[pl.ds(sid * ROWS_PER_TILE, ROWS_PER_TILE)],
        deg_hbm.at[cid, pl.ds(sid * ROWS_PER_TILE, ROWS_PER_TILE)],
    )


@jax.jit
def _sc_deg(dstp):
    return pl.kernel(
        _deg_body,
        out_type=jax.ShapeDtypeStruct((NC, NP), jnp.float32),
        mesh=_sc_mesh(),
        scratch_types=[
            pltpu.VMEM((CH,), jnp.int32),
            pltpu.VMEM((CH,), jnp.float32),
            pltpu.VMEM((CH,), jnp.float32),
            pltpu.VMEM_SHARED((NP,), jnp.float32),
        ],
    )(dstp)


def _msg_body(src_hbm, dst_hbm, h_hbm, acc_hbm, gi_v, si_v, rows_v, acc_sh):
    cid = lax.axis_index("c")
    sid = lax.axis_index("s")
    wid = cid * NS + sid

    def zrow(i, carry):
        rows_v[i, pl.ds(0, 16)] = jnp.zeros((16,), jnp.float32)
        rows_v[i, pl.ds(16, 16)] = jnp.zeros((16,), jnp.float32)
        return carry

    lax.fori_loop(0, CH, zrow, 0)
    for i in range(ROWS_PER_TILE // CH):
        pltpu.sync_copy(
            rows_v, acc_sh.at[pl.ds(sid * ROWS_PER_TILE + i * CH, CH)]
        )
    plsc.subcore_barrier()

    def chunk(j, carry):
        base = wid * PT + j * CH
        pltpu.sync_copy(src_hbm.at[pl.ds(base, CH)], gi_v)
        pltpu.sync_copy(dst_hbm.at[pl.ds(base, CH)], si_v)
        pltpu.sync_copy(h_hbm.at[gi_v], rows_v)
        pltpu.sync_copy(rows_v, acc_sh.at[si_v], add=True)
        return carry

    lax.fori_loop(0, NCHUNK, chunk, 0)
    plsc.subcore_barrier()
    pltpu.sync_copy(
        acc_sh.at[pl.ds(sid * ROWS_PER_TILE, ROWS_PER_TILE)],
        acc_hbm.at[cid, pl.ds(sid * ROWS_PER_TILE, ROWS_PER_TILE)],
    )


@jax.jit
def _sc_msg(srcp, dstp, h):
    return pl.kernel(
        _msg_body,
        out_type=jax.ShapeDtypeStruct((NC, NP, H), jnp.float32),
        mesh=_sc_mesh(),
        compiler_params=pltpu.CompilerParams(use_tc_tiling_on_sc=False),
        scratch_types=[
            pltpu.VMEM((CH,), jnp.int32),
            pltpu.VMEM((CH,), jnp.int32),
            pltpu.VMEM((CH, H), jnp.float32),
            pltpu.VMEM_SHARED((NP, H), jnp.float32),
        ],
    )(srcp, dstp, h)


def _tca_body(deg0_ref, deg1_ref, x_ref, w1_ref, dinvb_ref, h1s_ref):
    deg = deg0_ref[...] + deg1_ref[...] + 1.0
    dinv = lax.rsqrt(deg)
    dinvb = jnp.broadcast_to(dinv, (NP, H))
    g1 = lax.dot_general(
        x_ref[...], w1_ref[...],
        dimension_numbers=(((1,), (1,)), ((), ())),
        preferred_element_type=jnp.float32,
    )
    dinvb_ref[...] = dinvb
    h1s_ref[...] = g1 * dinvb


@jax.jit
def _tc_a(deg0, deg1, xp, w1):
    return pl.pallas_call(
        _tca_body,
        out_shape=(
            jax.ShapeDtypeStruct((NP, H), jnp.float32),
            jax.ShapeDtypeStruct((NP, H), jnp.float32),
        ),
    )(deg0, deg1, xp, w1)


def _tcb_body(a0_ref, a1_ref, h1s_ref, dinvb_ref, b1_ref, w2_ref, h2s_ref):
    dinvb = dinvb_ref[...]
    pre = dinvb * (a0_ref[...] + a1_ref[...] + h1s_ref[...]) + b1_ref[...]
    act = jnp.tanh(pre)
    g2 = lax.dot_general(
        act, w2_ref[...],
        dimension_numbers=(((1,), (1,)), ((), ())),
        preferred_element_type=jnp.float32,
    )
    h2s_ref[...] = g2 * dinvb


@jax.jit
def _tc_b(a0, a1, h1s, dinvb, b1r, w2):
    return pl.pallas_call(
        _tcb_body,
        out_shape=jax.ShapeDtypeStruct((NP, H), jnp.float32),
    )(a0, a1, h1s, dinvb, b1r, w2)


NBLK = 10
BLK = N // NBLK


def _tcc_body(a0_ref, a1_ref, h2s_ref, dinvb_ref, b2_ref, lw1_ref, lb1_ref,
              lw2_ref, lb2_ref, out_ref, pooled_ref):
    i = pl.program_id(0)

    @pl.when(i == 0)
    def _():
        pooled_ref[...] = jnp.zeros((1, H), jnp.float32)

    pre = dinvb_ref[...] * (a0_ref[...] + a1_ref[...] + h2s_ref[...]) + b2_ref[...]
    act = jnp.tanh(pre)
    g3 = lax.dot_general(
        act, lw1_ref[...],
        dimension_numbers=(((1,), (1,)), ((), ())),
        preferred_element_type=jnp.float32,
    )
    h3 = jnp.tanh(g3 + lb1_ref[...])
    pooled_ref[...] += jnp.sum(h3, axis=0, keepdims=True)

    @pl.when(i == NBLK - 1)
    def _():
        out_ref[...] = (
            jnp.sum(pooled_ref[...] * lw2_ref[...], axis=1, keepdims=True)
            + lb2_ref[...]
        )


@jax.jit
def _tc_c(a0, a1, h2s, dinvb, b2r, lw1, lb1r, lw2r, lb2r):
    row_spec = pl.BlockSpec((BLK, H), lambda i: (i, 0))
    small = lambda shp: pl.BlockSpec(shp, lambda i: (0, 0))
    return pl.pallas_call(
        _tcc_body,
        grid=(NBLK,),
        in_specs=[
            row_spec, row_spec, row_spec, row_spec,
            small((1, H)), small((H, H)), small((1, H)),
            small((1, H)), small((1, 1)),
        ],
        out_specs=small((1, 1)),
        out_shape=jax.ShapeDtypeStruct((1, 1), jnp.float32),
        scratch_shapes=[pltpu.VMEM((1, H), jnp.float32)],
    )(a0, a1, h2s, dinvb, b2r, lw1, lb1r, lw2r, lb2r)


def kernel(x, edge_index, W1, b1, W2, b2, LW1, Lb1, LW2, Lb2):
    pad = EP - E
    padidx = (N + (jnp.arange(pad, dtype=jnp.int32) % (NP - N))).astype(jnp.int32)
    srcp = jnp.concatenate([edge_index[0].astype(jnp.int32), padidx])
    dstp = jnp.concatenate([edge_index[1].astype(jnp.int32), padidx])
    xp = jnp.pad(x, ((0, NP - N), (0, 0)))

    degp = _sc_deg(dstp)
    deg0 = degp[0].reshape(NP, 1)
    deg1 = degp[1].reshape(NP, 1)

    dinvb, h1s = _tc_a(deg0, deg1, xp, W1)
    acc1 = _sc_msg(srcp, dstp, h1s)
    h2s = _tc_b(acc1[0], acc1[1], h1s, dinvb, b1.reshape(1, H), W2)
    acc2 = _sc_msg(srcp, dstp, h2s)
    out = _tc_c(
        acc2[0], acc2[1], h2s, dinvb,
        b2.reshape(1, H), LW1, Lb1.reshape(1, H),
        LW2.reshape(1, H), Lb2.reshape(1, 1),
    )
    return out.reshape(1)


# speedup vs baseline: 54.1803x; 2.6492x over previous
import jax
import jax.numpy as jnp
from jax import lax
from jax.experimental import pallas as pl
from jax.experimental.pallas import tpu as pltpu
from jax.experimental.pallas import tpu_sc as plsc

N = 10000
D = 128
E = 320000
H = 32

NC = 2
NS = 16
NW = NC * NS

NP = 10240
CH = 128
CROWS = E // CH
RB = CROWS // NW
REXTRA = CROWS - RB * NW
RMAX = RB + 1
NBUF = 4
NRND = RB // NBUF
ROWS_PER_TILE = NP // NS


def _sc_mesh():
    return plsc.VectorSubcoreMesh(core_axis_name="c", subcore_axis_name="s")


def _worker_rows(wid):
    base_row = wid * RB + jnp.minimum(wid, REXTRA)
    nrows = jnp.where(wid < REXTRA, RB + 1, RB)
    return base_row, nrows


def _preload_idx(e2d, row, base_row, dst_v):
    pltpu.sync_copy(e2d.at[row, pl.ds(base_row, RB)], dst_v.at[pl.ds(0, RB)])


def _preload_extra(e2d, row, base_row, dst_v, wid):
    @pl.when(wid < REXTRA)
    def _():
        pltpu.sync_copy(
            e2d.at[row, pl.ds(base_row + RB, 1)], dst_v.at[pl.ds(RB, 1)]
        )


def _deg_body(e2d, deg_hbm, didx, ones_v, zv, deg_sh, sem):
    cid = lax.axis_index("c")
    sid = lax.axis_index("s")
    wid = cid * NS + sid
    base_row, nrows = _worker_rows(wid)

    _preload_idx(e2d, 1, base_row, didx)
    _preload_extra(e2d, 1, base_row, didx, wid)
    for k in range(CH // 16):
        ones_v[pl.ds(16 * k, 16)] = jnp.full((16,), 1.0, jnp.float32)
        zv[pl.ds(16 * k, 16)] = jnp.zeros((16,), jnp.float32)
    for i in range(ROWS_PER_TILE // CH):
        pltpu.sync_copy(zv, deg_sh.at[pl.ds(sid * ROWS_PER_TILE + i * CH, CH)])
    plsc.subcore_barrier()

    def rnd(r, carry):
        for b in range(NBUF):
            pltpu.async_copy(ones_v, deg_sh.at[didx.at[r * NBUF + b]],
                             sem, add=True)
        for b in range(NBUF):
            pltpu.make_async_copy(
                ones_v, deg_sh.at[didx.at[r * NBUF + b]], sem).wait()
        return carry

    lax.fori_loop(0, NRND, rnd, 0)
    for t in range(NRND * NBUF, RMAX):
        @pl.when(t < nrows)
        def _():
            pltpu.sync_copy(ones_v, deg_sh.at[didx.at[t]], add=True)

    plsc.subcore_barrier()
    pltpu.sync_copy(
        deg_sh.at[pl.ds(sid * ROWS_PER_TILE, ROWS_PER_TILE)],
        deg_hbm.at[cid, pl.ds(sid * ROWS_PER_TILE, ROWS_PER_TILE)],
    )


@jax.jit
def _sc_deg(e2d):
    return pl.kernel(
        _deg_body,
        out_type=jax.ShapeDtypeStruct((NC, NP), jnp.float32),
        mesh=_sc_mesh(),
        compiler_params=pltpu.CompilerParams(use_tc_tiling_on_sc=False),
        scratch_types=[
            pltpu.VMEM((RMAX, CH), jnp.int32),
            pltpu.VMEM((CH,), jnp.float32),
            pltpu.VMEM((CH,), jnp.float32),
            pltpu.VMEM_SHARED((NP,), jnp.float32),
            pltpu.SemaphoreType.DMA,
        ],
    )(e2d)


def _msg_body(e2d, h_hbm, acc_hbm, sidx, didx, rows_v, acc_sh, sem_g, sem_s):
    cid = lax.axis_index("c")
    sid = lax.axis_index("s")
    wid = cid * NS + sid
    base_row, nrows = _worker_rows(wid)

    _preload_idx(e2d, 0, base_row, sidx)
    _preload_idx(e2d, 1, base_row, didx)
    _preload_extra(e2d, 0, base_row, sidx, wid)
    _preload_extra(e2d, 1, base_row, didx, wid)

    def zrow(i, carry):
        rows_v[0, i, pl.ds(0, 16)] = jnp.zeros((16,), jnp.float32)
        rows_v[0, i, pl.ds(16, 16)] = jnp.zeros((16,), jnp.float32)
        return carry

    lax.fori_loop(0, CH, zrow, 0)
    for i in range(ROWS_PER_TILE // CH):
        pltpu.sync_copy(
            rows_v.at[0], acc_sh.at[pl.ds(sid * ROWS_PER_TILE + i * CH, CH)]
        )
    plsc.subcore_barrier()

    def g_start(j, s):
        pltpu.async_copy(h_hbm.at[sidx.at[j]], rows_v.at[s], sem_g.at[s])

    def g_wait(j, s):
        pltpu.make_async_copy(
            h_hbm.at[sidx.at[j]], rows_v.at[s], sem_g.at[s]).wait()

    def s_start(j, s):
        pltpu.async_copy(rows_v.at[s], acc_sh.at[didx.at[j]],
                         sem_s.at[s], add=True)

    def s_wait(j, s):
        pltpu.make_async_copy(
            rows_v.at[s], acc_sh.at[didx.at[j]], sem_s.at[s]).wait()

    for b in range(NBUF):
        g_start(b, b)

    def rnd(r, carry):
        g = (r % 2) * NBUF
        gn = ((r + 1) % 2) * NBUF
        for b in range(NBUF):
            j = r * NBUF + b
            g_wait(j, g + b)
            s_start(j, g + b)
        for b in range(NBUF):
            jn = (r + 1) * NBUF + b

            @pl.when(r >= 1)
            def _():
                s_wait((r - 1) * NBUF + b, gn + b)

            @pl.when(jn < nrows)
            def _():
                g_start(jn, gn + b)
        return carry

    lax.fori_loop(0, NRND, rnd, 0)

    gt = (NRND % 2) * NBUF
    go = ((NRND - 1) % 2) * NBUF
    for bt in range(NRND * NBUF, RMAX):
        b = bt - NRND * NBUF

        @pl.when(bt < nrows)
        def _():
            g_wait(bt, gt + b)
            s_start(bt, gt + b)

    for b in range(NBUF):
        s_wait((NRND - 1) * NBUF + b, go + b)
    for bt in range(NRND * NBUF, RMAX):
        b = bt - NRND * NBUF

        @pl.when(bt < nrows)
        def _():
            s_wait(bt, gt + b)

    plsc.subcore_barrier()
    pltpu.sync_copy(
        acc_sh.at[pl.ds(sid * ROWS_PER_TILE, ROWS_PER_TILE)],
        acc_hbm.at[cid, pl.ds(sid * ROWS_PER_TILE, ROWS_PER_TILE)],
    )


@jax.jit
def _sc_msg(e2d, h):
    return pl.kernel(
        _msg_body,
        out_type=jax.ShapeDtypeStruct((NC, NP, H), jnp.float32),
        mesh=_sc_mesh(),
        compiler_params=pltpu.CompilerParams(use_tc_tiling_on_sc=False),
        scratch_types=[
            pltpu.VMEM((RMAX, CH), jnp.int32),
            pltpu.VMEM((RMAX, CH), jnp.int32),
            pltpu.VMEM((2 * NBUF, CH, H), jnp.float32),
            pltpu.VMEM_SHARED((NP, H), jnp.float32),
            pltpu.SemaphoreType.DMA((2 * NBUF,)),
            pltpu.SemaphoreType.DMA((2 * NBUF,)),
        ],
    )(e2d, h)


def _tca_body(deg0_ref, deg1_ref, x_ref, w1_ref, dinvb_ref, h1s_ref):
    deg = deg0_ref[...] + deg1_ref[...] + 1.0
    dinv = lax.rsqrt(deg)
    dinvb = jnp.broadcast_to(dinv, (NP, H))
    g1 = lax.dot_general(
        x_ref[...], w1_ref[...],
        dimension_numbers=(((1,), (1,)), ((), ())),
        preferred_element_type=jnp.float32,
    )
    dinvb_ref[...] = dinvb
    h1s_ref[pl.ds(0, N), :] = g1 * dinvb[:N]
    h1s_ref[pl.ds(N, NP - N), :] = jnp.zeros((NP - N, H), jnp.float32)


@jax.jit
def _tc_a(deg0, deg1, x, w1):
    return pl.pallas_call(
        _tca_body,
        out_shape=(
            jax.ShapeDtypeStruct((NP, H), jnp.float32),
            jax.ShapeDtypeStruct((NP, H), jnp.float32),
        ),
    )(deg0, deg1, x, w1)


def _tcb_body(a0_ref, a1_ref, h1s_ref, dinvb_ref, b1_ref, w2_ref, h2s_ref):
    dinvb = dinvb_ref[...]
    pre = dinvb * (a0_ref[...] + a1_ref[...] + h1s_ref[...]) + b1_ref[...]
    act = jnp.tanh(pre)
    g2 = lax.dot_general(
        act, w2_ref[...],
        dimension_numbers=(((1,), (1,)), ((), ())),
        preferred_element_type=jnp.float32,
    )
    h2s_ref[...] = g2 * dinvb


@jax.jit
def _tc_b(a0, a1, h1s, dinvb, b1r, w2):
    return pl.pallas_call(
        _tcb_body,
        out_shape=jax.ShapeDtypeStruct((NP, H), jnp.float32),
    )(a0, a1, h1s, dinvb, b1r, w2)


NBLK = 10
BLK = N // NBLK


def _tcc_body(a0_ref, a1_ref, h2s_ref, dinvb_ref, b2_ref, lw1_ref, lb1_ref,
              lw2_ref, lb2_ref, out_ref, pooled_ref):
    i = pl.program_id(0)

    @pl.when(i == 0)
    def _():
        pooled_ref[...] = jnp.zeros((1, H), jnp.float32)

    pre = dinvb_ref[...] * (a0_ref[...] + a1_ref[...] + h2s_ref[...]) + b2_ref[...]
    act = jnp.tanh(pre)
    g3 = lax.dot_general(
        act, lw1_ref[...],
        dimension_numbers=(((1,), (1,)), ((), ())),
        preferred_element_type=jnp.float32,
    )
    h3 = jnp.tanh(g3 + lb1_ref[...])
    pooled_ref[...] += jnp.sum(h3, axis=0, keepdims=True)

    @pl.when(i == NBLK - 1)
    def _():
        out_ref[...] = (
            jnp.sum(pooled_ref[...] * lw2_ref[...], axis=1, keepdims=True)
            + lb2_ref[...]
        )


@jax.jit
def _tc_c(a0, a1, h2s, dinvb, b2r, lw1, lb1r, lw2r, lb2r):
    row_spec = pl.BlockSpec((BLK, H), lambda i: (i, 0))
    small = lambda shp: pl.BlockSpec(shp, lambda i: (0, 0))
    return pl.pallas_call(
        _tcc_body,
        grid=(NBLK,),
        in_specs=[
            row_spec, row_spec, row_spec, row_spec,
            small((1, H)), small((H, H)), small((1, H)),
            small((1, H)), small((1, 1)),
        ],
        out_specs=small((1, 1)),
        out_shape=jax.ShapeDtypeStruct((1, 1), jnp.float32),
        scratch_shapes=[pltpu.VMEM((1, H), jnp.float32)],
    )(a0, a1, h2s, dinvb, b2r, lw1, lb1r, lw2r, lb2r)


def kernel(x, edge_index, W1, b1, W2, b2, LW1, Lb1, LW2, Lb2):
    e2d = edge_index.astype(jnp.int32).reshape(2, CROWS, CH)

    degp = _sc_deg(e2d)
    deg0 = degp[0].reshape(NP, 1)
    deg1 = degp[1].reshape(NP, 1)

    dinvb, h1s = _tc_a(deg0, deg1, x, W1)
    acc1 = _sc_msg(e2d, h1s)
    h2s = _tc_b(acc1[0], acc1[1], h1s, dinvb, b1.reshape(1, H), W2)
    acc2 = _sc_msg(e2d, h2s)
    out = _tc_c(
        acc2[0], acc2[1], h2s, dinvb,
        b2.reshape(1, H), LW1, Lb1.reshape(1, H),
        LW2.reshape(1, H), Lb2.reshape(1, 1),
    )
    return out.reshape(1)


# speedup vs baseline: 57.0989x; 1.0539x over previous
import jax
import jax.numpy as jnp
from jax import lax
from jax.experimental import pallas as pl
from jax.experimental.pallas import tpu as pltpu
from jax.experimental.pallas import tpu_sc as plsc

N = 10000
D = 128
E = 320000
H = 32

NC = 2
NS = 16
NW = NC * NS

NP = 10240
CH = 128
CROWS = E // CH
RB = CROWS // NW
REXTRA = CROWS - RB * NW
RMAX = RB + 1
NBUF = 4
NRND = RB // NBUF
ROWS_PER_TILE = NP // NS


def _sc_mesh():
    return plsc.VectorSubcoreMesh(core_axis_name="c", subcore_axis_name="s")


def _worker_rows(wid):
    base_row = wid * RB + jnp.minimum(wid, REXTRA)
    nrows = jnp.where(wid < REXTRA, RB + 1, RB)
    return base_row, nrows


def _preload_idx(e2d, row, base_row, dst_v):
    pltpu.sync_copy(e2d.at[row, pl.ds(base_row, RB)], dst_v.at[pl.ds(0, RB)])


def _preload_extra(e2d, row, base_row, dst_v, wid):
    @pl.when(wid < REXTRA)
    def _():
        pltpu.sync_copy(
            e2d.at[row, pl.ds(base_row + RB, 1)], dst_v.at[pl.ds(RB, 1)]
        )


def _deg_body(e2d, degb_hbm, didx, ones_v, zv, deg_v, degb_v, deg_sh, sem):
    cid = lax.axis_index("c")
    sid = lax.axis_index("s")
    wid = cid * NS + sid
    base_row, nrows = _worker_rows(wid)

    _preload_idx(e2d, 1, base_row, didx)
    _preload_extra(e2d, 1, base_row, didx, wid)
    for k in range(CH // 16):
        ones_v[pl.ds(16 * k, 16)] = jnp.full((16,), 1.0, jnp.float32)
        zv[pl.ds(16 * k, 16)] = jnp.zeros((16,), jnp.float32)
    for i in range(ROWS_PER_TILE // CH):
        pltpu.sync_copy(zv, deg_sh.at[pl.ds(sid * ROWS_PER_TILE + i * CH, CH)])
    plsc.subcore_barrier()

    def rnd(r, carry):
        for b in range(NBUF):
            pltpu.async_copy(ones_v, deg_sh.at[didx.at[r * NBUF + b]],
                             sem, add=True)
        for b in range(NBUF):
            pltpu.make_async_copy(
                ones_v, deg_sh.at[didx.at[r * NBUF + b]], sem).wait()
        return carry

    lax.fori_loop(0, NRND, rnd, 0)
    for t in range(NRND * NBUF, RMAX):
        @pl.when(t < nrows)
        def _():
            pltpu.sync_copy(ones_v, deg_sh.at[didx.at[t]], add=True)

    plsc.subcore_barrier()
    pltpu.sync_copy(
        deg_sh.at[pl.ds(sid * ROWS_PER_TILE, ROWS_PER_TILE)], deg_v
    )

    def brow(r, carry):
        row = plsc.load_gather(deg_v, [jnp.full((16,), r, jnp.int32)])
        for k in range(H // 16):
            degb_v[r, pl.ds(16 * k, 16)] = row
        return carry

    lax.fori_loop(0, ROWS_PER_TILE, brow, 0)
    pltpu.sync_copy(
        degb_v, degb_hbm.at[cid, pl.ds(sid * ROWS_PER_TILE, ROWS_PER_TILE)]
    )


@jax.jit
def _sc_deg(e2d):
    return pl.kernel(
        _deg_body,
        out_type=jax.ShapeDtypeStruct((NC, NP, H), jnp.float32),
        mesh=_sc_mesh(),
        compiler_params=pltpu.CompilerParams(
            use_tc_tiling_on_sc=False, needs_layout_passes=False),
        scratch_types=[
            pltpu.VMEM((RMAX, CH), jnp.int32),
            pltpu.VMEM((CH,), jnp.float32),
            pltpu.VMEM((CH,), jnp.float32),
            pltpu.VMEM((ROWS_PER_TILE,), jnp.float32),
            pltpu.VMEM((ROWS_PER_TILE, H), jnp.float32),
            pltpu.VMEM_SHARED((NP,), jnp.float32),
            pltpu.SemaphoreType.DMA,
        ],
    )(e2d)


def _msg_body(e2d, h_hbm, acc_hbm, sidx, didx, rows_v, acc_sh, sem_g, sem_s):
    cid = lax.axis_index("c")
    sid = lax.axis_index("s")
    wid = cid * NS + sid
    base_row, nrows = _worker_rows(wid)

    _preload_idx(e2d, 0, base_row, sidx)
    _preload_idx(e2d, 1, base_row, didx)
    _preload_extra(e2d, 0, base_row, sidx, wid)
    _preload_extra(e2d, 1, base_row, didx, wid)

    def zrow(i, carry):
        rows_v[0, i, pl.ds(0, 16)] = jnp.zeros((16,), jnp.float32)
        rows_v[0, i, pl.ds(16, 16)] = jnp.zeros((16,), jnp.float32)
        return carry

    lax.fori_loop(0, CH, zrow, 0)
    for i in range(ROWS_PER_TILE // CH):
        pltpu.sync_copy(
            rows_v.at[0], acc_sh.at[pl.ds(sid * ROWS_PER_TILE + i * CH, CH)]
        )
    plsc.subcore_barrier()

    def g_start(j, s):
        pltpu.async_copy(h_hbm.at[sidx.at[j]], rows_v.at[s], sem_g.at[s])

    def g_wait(j, s):
        pltpu.make_async_copy(
            h_hbm.at[sidx.at[j]], rows_v.at[s], sem_g.at[s]).wait()

    def s_start(j, s):
        pltpu.async_copy(rows_v.at[s], acc_sh.at[didx.at[j]],
                         sem_s.at[s], add=True)

    def s_wait(j, s):
        pltpu.make_async_copy(
            rows_v.at[s], acc_sh.at[didx.at[j]], sem_s.at[s]).wait()

    for b in range(NBUF):
        g_start(b, b)

    def rnd(r, carry):
        g = (r % 2) * NBUF
        gn = ((r + 1) % 2) * NBUF
        for b in range(NBUF):
            j = r * NBUF + b
            g_wait(j, g + b)
            s_start(j, g + b)
        for b in range(NBUF):
            jn = (r + 1) * NBUF + b

            @pl.when(r >= 1)
            def _():
                s_wait((r - 1) * NBUF + b, gn + b)

            @pl.when(jn < nrows)
            def _():
                g_start(jn, gn + b)
        return carry

    lax.fori_loop(0, NRND, rnd, 0)

    gt = (NRND % 2) * NBUF
    go = ((NRND - 1) % 2) * NBUF
    for bt in range(NRND * NBUF, RMAX):
        b = bt - NRND * NBUF

        @pl.when(bt < nrows)
        def _():
            g_wait(bt, gt + b)
            s_start(bt, gt + b)

    for b in range(NBUF):
        s_wait((NRND - 1) * NBUF + b, go + b)
    for bt in range(NRND * NBUF, RMAX):
        b = bt - NRND * NBUF

        @pl.when(bt < nrows)
        def _():
            s_wait(bt, gt + b)

    plsc.subcore_barrier()
    pltpu.sync_copy(
        acc_sh.at[pl.ds(sid * ROWS_PER_TILE, ROWS_PER_TILE)],
        acc_hbm.at[cid, pl.ds(sid * ROWS_PER_TILE, ROWS_PER_TILE)],
    )


@jax.jit
def _sc_msg(e2d, h):
    return pl.kernel(
        _msg_body,
        out_type=jax.ShapeDtypeStruct((NC, NP, H), jnp.float32),
        mesh=_sc_mesh(),
        compiler_params=pltpu.CompilerParams(use_tc_tiling_on_sc=False),
        scratch_types=[
            pltpu.VMEM((RMAX, CH), jnp.int32),
            pltpu.VMEM((RMAX, CH), jnp.int32),
            pltpu.VMEM((2 * NBUF, CH, H), jnp.float32),
            pltpu.VMEM_SHARED((NP, H), jnp.float32),
            pltpu.SemaphoreType.DMA((2 * NBUF,)),
            pltpu.SemaphoreType.DMA((2 * NBUF,)),
        ],
    )(e2d, h)


def _tca_body(degb_ref, x_ref, w1_ref, dinvb_ref, h1s_ref):
    dinvb = lax.rsqrt(degb_ref[0] + degb_ref[1] + 1.0)
    g1 = lax.dot_general(
        x_ref[...], w1_ref[...],
        dimension_numbers=(((1,), (1,)), ((), ())),
        preferred_element_type=jnp.float32,
    )
    dinvb_ref[...] = dinvb
    h1s_ref[pl.ds(0, N), :] = g1 * dinvb[:N]
    h1s_ref[pl.ds(N, NP - N), :] = jnp.zeros((NP - N, H), jnp.float32)


@jax.jit
def _tc_a(degb, x, w1):
    return pl.pallas_call(
        _tca_body,
        out_shape=(
            jax.ShapeDtypeStruct((NP, H), jnp.float32),
            jax.ShapeDtypeStruct((NP, H), jnp.float32),
        ),
    )(degb, x, w1)


def _tcb_body(acc_ref, h1s_ref, dinvb_ref, b1_ref, w2_ref, h2s_ref):
    dinvb = dinvb_ref[...]
    pre = (dinvb * (acc_ref[0] + acc_ref[1] + h1s_ref[...])
           + b1_ref[...][None, :])
    act = jnp.tanh(pre)
    g2 = lax.dot_general(
        act, w2_ref[...],
        dimension_numbers=(((1,), (1,)), ((), ())),
        preferred_element_type=jnp.float32,
    )
    h2s_ref[...] = g2 * dinvb


@jax.jit
def _tc_b(acc, h1s, dinvb, b1, w2):
    return pl.pallas_call(
        _tcb_body,
        out_shape=jax.ShapeDtypeStruct((NP, H), jnp.float32),
    )(acc, h1s, dinvb, b1, w2)


NBLK = 10
BLK = N // NBLK


def _tcc_body(acc_ref, h2s_ref, dinvb_ref, b2_ref, lw1_ref, lb1_ref,
              lw2_ref, lb2_ref, out_ref, pooled_ref):
    i = pl.program_id(0)

    @pl.when(i == 0)
    def _():
        pooled_ref[...] = jnp.zeros((1, H), jnp.float32)

    pre = (dinvb_ref[...] * (acc_ref[0] + acc_ref[1] + h2s_ref[...])
           + b2_ref[...][None, :])
    act = jnp.tanh(pre)
    g3 = lax.dot_general(
        act, lw1_ref[...],
        dimension_numbers=(((1,), (1,)), ((), ())),
        preferred_element_type=jnp.float32,
    )
    h3 = jnp.tanh(g3 + lb1_ref[...][None, :])
    pooled_ref[...] += jnp.sum(h3, axis=0, keepdims=True)

    @pl.when(i == NBLK - 1)
    def _():
        out_ref[...] = (
            jnp.sum(pooled_ref[...] * lw2_ref[...], axis=1, keepdims=True)
            + lb2_ref[...][None, :]
        )


@jax.jit
def _tc_c(acc, h2s, dinvb, b2, lw1, lb1, lw2, lb2):
    row_spec = pl.BlockSpec((BLK, H), lambda i: (i, 0))
    acc_spec = pl.BlockSpec((2, BLK, H), lambda i: (0, i, 0))
    small1 = pl.BlockSpec((H,), lambda i: (0,))
    return pl.pallas_call(
        _tcc_body,
        grid=(NBLK,),
        in_specs=[
            acc_spec, row_spec, row_spec,
            small1, pl.BlockSpec((H, H), lambda i: (0, 0)), small1,
            pl.BlockSpec((1, H), lambda i: (0, 0)),
            pl.BlockSpec((1,), lambda i: (0,)),
        ],
        out_specs=pl.BlockSpec((1, 1), lambda i: (0, 0)),
        out_shape=jax.ShapeDtypeStruct((1, 1), jnp.float32),
        scratch_shapes=[pltpu.VMEM((1, H), jnp.float32)],
    )(acc, h2s, dinvb, b2, lw1, lb1, lw2, lb2)


def kernel(x, edge_index, W1, b1, W2, b2, LW1, Lb1, LW2, Lb2):
    e2d = edge_index.astype(jnp.int32).reshape(2, CROWS, CH)

    degb = _sc_deg(e2d)
    dinvb, h1s = _tc_a(degb, x, W1)
    acc1 = _sc_msg(e2d, h1s)
    h2s = _tc_b(acc1, h1s, dinvb, b1, W2)
    acc2 = _sc_msg(e2d, h2s)
    out = _tc_c(acc2, h2s, dinvb, b2, LW1, Lb1, LW2, Lb2)
    return out.reshape(1)


# speedup vs baseline: 80.4257x; 1.4085x over previous
import jax
import jax.numpy as jnp
from jax import lax
from jax.experimental import pallas as pl
from jax.experimental.pallas import tpu as pltpu
from jax.experimental.pallas import tpu_sc as plsc

N = 10000
D = 128
E = 320000
H = 32

NC = 2
NS = 16
NW = NC * NS

NP = 10240
CH = 128
CROWS = E // CH
RB = CROWS // NW
REXTRA = CROWS - RB * NW
RMAX = RB + 1
NBUF = 4
NRND = RB // NBUF
ROWS_PER_TILE = NP // NS


def _sc_mesh():
    return plsc.VectorSubcoreMesh(core_axis_name="c", subcore_axis_name="s")


def _worker_rows(wid):
    base_row = wid * RB + jnp.minimum(wid, REXTRA)
    nrows = jnp.where(wid < REXTRA, RB + 1, RB)
    return base_row, nrows


def _preload_idx(e2d, row, base_row, dst_v):
    pltpu.sync_copy(e2d.at[row, pl.ds(base_row, RB)], dst_v.at[pl.ds(0, RB)])


def _preload_extra(e2d, row, base_row, dst_v, wid):
    @pl.when(wid < REXTRA)
    def _():
        pltpu.sync_copy(
            e2d.at[row, pl.ds(base_row + RB, 1)], dst_v.at[pl.ds(RB, 1)]
        )


def _deg_body(e2d, degb_hbm, didx, ones_v, zv, deg_v, degb_v, deg_sh, sem):
    cid = lax.axis_index("c")
    sid = lax.axis_index("s")
    wid = cid * NS + sid
    base_row, nrows = _worker_rows(wid)

    _preload_idx(e2d, 1, base_row, didx)
    _preload_extra(e2d, 1, base_row, didx, wid)
    for k in range(CH // 16):
        ones_v[pl.ds(16 * k, 16)] = jnp.full((16,), 1.0, jnp.float32)
        zv[pl.ds(16 * k, 16)] = jnp.zeros((16,), jnp.float32)
    for i in range(ROWS_PER_TILE // CH):
        pltpu.sync_copy(zv, deg_sh.at[pl.ds(sid * ROWS_PER_TILE + i * CH, CH)])
    plsc.subcore_barrier()

    def rnd(r, carry):
        for b in range(NBUF):
            pltpu.async_copy(ones_v, deg_sh.at[didx.at[r * NBUF + b]],
                             sem, add=True)
        for b in range(NBUF):
            pltpu.make_async_copy(
                ones_v, deg_sh.at[didx.at[r * NBUF + b]], sem).wait()
        return carry

    lax.fori_loop(0, NRND, rnd, 0)
    for t in range(NRND * NBUF, RMAX):
        @pl.when(t < nrows)
        def _():
            pltpu.sync_copy(ones_v, deg_sh.at[didx.at[t]], add=True)

    plsc.subcore_barrier()
    pltpu.sync_copy(
        deg_sh.at[pl.ds(sid * ROWS_PER_TILE, ROWS_PER_TILE)], deg_v
    )

    def brow(r, carry):
        row = plsc.load_gather(deg_v, [jnp.full((16,), r, jnp.int32)])
        for k in range(H // 16):
            degb_v[r, pl.ds(16 * k, 16)] = row
        return carry

    lax.fori_loop(0, ROWS_PER_TILE, brow, 0)
    pltpu.sync_copy(
        degb_v, degb_hbm.at[cid, pl.ds(sid * ROWS_PER_TILE, ROWS_PER_TILE)]
    )


@jax.jit
def _sc_deg(e2d):
    return pl.kernel(
        _deg_body,
        out_type=jax.ShapeDtypeStruct((NC, NP, H), jnp.float32),
        mesh=_sc_mesh(),
        compiler_params=pltpu.CompilerParams(
            use_tc_tiling_on_sc=False, needs_layout_passes=False),
        scratch_types=[
            pltpu.VMEM((RMAX, CH), jnp.int32),
            pltpu.VMEM((CH,), jnp.float32),
            pltpu.VMEM((CH,), jnp.float32),
            pltpu.VMEM((ROWS_PER_TILE,), jnp.float32),
            pltpu.VMEM((ROWS_PER_TILE, H), jnp.float32),
            pltpu.VMEM_SHARED((NP,), jnp.float32),
            pltpu.SemaphoreType.DMA,
        ],
    )(e2d)


def _msg_body(e2d, h_hbm, acc_hbm, sidx, didx, rows_v, acc_sh, sem_g, sem_s):
    cid = lax.axis_index("c")
    sid = lax.axis_index("s")
    wid = cid * NS + sid
    base_row, nrows = _worker_rows(wid)

    _preload_idx(e2d, 0, base_row, sidx)
    _preload_idx(e2d, 1, base_row, didx)
    _preload_extra(e2d, 0, base_row, sidx, wid)
    _preload_extra(e2d, 1, base_row, didx, wid)

    def zrow(i, carry):
        rows_v[0, i, pl.ds(0, 16)] = jnp.zeros((16,), jnp.float32)
        rows_v[0, i, pl.ds(16, 16)] = jnp.zeros((16,), jnp.float32)
        return carry

    lax.fori_loop(0, CH, zrow, 0)
    for i in range(ROWS_PER_TILE // CH):
        pltpu.sync_copy(
            rows_v.at[0], acc_sh.at[pl.ds(sid * ROWS_PER_TILE + i * CH, CH)]
        )
    plsc.subcore_barrier()

    def g_start(j, s):
        pltpu.async_copy(h_hbm.at[sidx.at[j]], rows_v.at[s], sem_g.at[s])

    def g_wait(j, s):
        pltpu.make_async_copy(
            h_hbm.at[sidx.at[j]], rows_v.at[s], sem_g.at[s]).wait()

    def s_start(j, s):
        pltpu.async_copy(rows_v.at[s], acc_sh.at[didx.at[j]],
                         sem_s.at[s], add=True)

    def s_wait(j, s):
        pltpu.make_async_copy(
            rows_v.at[s], acc_sh.at[didx.at[j]], sem_s.at[s]).wait()

    for b in range(NBUF):
        g_start(b, b)

    def rnd(r, carry):
        g = (r % 2) * NBUF
        gn = ((r + 1) % 2) * NBUF
        for b in range(NBUF):
            j = r * NBUF + b
            g_wait(j, g + b)
            s_start(j, g + b)
        for b in range(NBUF):
            jn = (r + 1) * NBUF + b

            @pl.when(r >= 1)
            def _():
                s_wait((r - 1) * NBUF + b, gn + b)

            @pl.when(jn < nrows)
            def _():
                g_start(jn, gn + b)
        return carry

    lax.fori_loop(0, NRND, rnd, 0)

    gt = (NRND % 2) * NBUF
    go = ((NRND - 1) % 2) * NBUF
    for bt in range(NRND * NBUF, RMAX):
        b = bt - NRND * NBUF

        @pl.when(bt < nrows)
        def _():
            g_wait(bt, gt + b)
            s_start(bt, gt + b)

    for b in range(NBUF):
        s_wait((NRND - 1) * NBUF + b, go + b)
    for bt in range(NRND * NBUF, RMAX):
        b = bt - NRND * NBUF

        @pl.when(bt < nrows)
        def _():
            s_wait(bt, gt + b)

    plsc.subcore_barrier()
    pltpu.sync_copy(
        acc_sh.at[pl.ds(sid * ROWS_PER_TILE, ROWS_PER_TILE)],
        acc_hbm.at[cid, pl.ds(sid * ROWS_PER_TILE, ROWS_PER_TILE)],
    )


@jax.jit
def _sc_msg(e2d, h):
    return pl.kernel(
        _msg_body,
        out_type=jax.ShapeDtypeStruct((NC, NP, H), jnp.float32),
        mesh=_sc_mesh(),
        compiler_params=pltpu.CompilerParams(use_tc_tiling_on_sc=False),
        scratch_types=[
            pltpu.VMEM((RMAX, CH), jnp.int32),
            pltpu.VMEM((RMAX, CH), jnp.int32),
            pltpu.VMEM((2 * NBUF, CH, H), jnp.float32),
            pltpu.VMEM_SHARED((NP, H), jnp.float32),
            pltpu.SemaphoreType.DMA((2 * NBUF,)),
            pltpu.SemaphoreType.DMA((2 * NBUF,)),
        ],
    )(e2d, h)


VR = NP // 4
VRN = N // 4


def _blockdiag(w):
    t1 = jnp.concatenate([w, w, w, w], axis=0)
    t2 = jnp.concatenate([t1, t1, t1, t1], axis=1)
    ri = lax.broadcasted_iota(jnp.int32, (4 * H, 4 * H), 0)
    ci = lax.broadcasted_iota(jnp.int32, (4 * H, 4 * H), 1)
    return jnp.where((ri // H) == (ci // H), t2, 0.0)


def _tile4(b):
    return jnp.concatenate([b, b, b, b], axis=0)


def _tca_body(degb_ref, xv_ref, w1_ref, dinvbv_ref, h1sv_ref):
    dinvbv = lax.rsqrt(degb_ref[0] + degb_ref[1] + 1.0)
    t1 = jnp.concatenate([w1_ref[...]] * 4, axis=0)
    t2 = jnp.concatenate([t1] * 4, axis=1)
    ri = lax.broadcasted_iota(jnp.int32, (4 * H, 4 * D), 0)
    ci = lax.broadcasted_iota(jnp.int32, (4 * H, 4 * D), 1)
    w1bd = jnp.where((ri // H) == (ci // D), t2, 0.0)
    g1v = lax.dot_general(
        xv_ref[...], w1bd,
        dimension_numbers=(((1,), (1,)), ((), ())),
        preferred_element_type=jnp.float32,
    )
    g1vf = jnp.concatenate(
        [g1v, jnp.zeros((VR - VRN, 4 * H), jnp.float32)], axis=0)
    dinvbv_ref[...] = dinvbv
    h1sv_ref[...] = g1vf * dinvbv


@jax.jit
def _tc_a(degb, xv, w1):
    return pl.pallas_call(
        _tca_body,
        out_shape=(
            jax.ShapeDtypeStruct((VR, 4 * H), jnp.float32),
            jax.ShapeDtypeStruct((VR, 4 * H), jnp.float32),
        ),
    )(degb, xv, w1)


def _tcb_body(acc_ref, h1s_ref, dinvb_ref, b1_ref, w2_ref, h2s_ref):
    dinvb = dinvb_ref[...]
    pre = (dinvb * (acc_ref[0] + acc_ref[1] + h1s_ref[...])
           + _tile4(b1_ref[...])[None, :])
    act = jnp.tanh(pre)
    g2 = lax.dot_general(
        act, _blockdiag(w2_ref[...]),
        dimension_numbers=(((1,), (1,)), ((), ())),
        preferred_element_type=jnp.float32,
    )
    h2s_ref[...] = g2 * dinvb


@jax.jit
def _tc_b(acc, h1s, dinvb, b1, w2):
    return pl.pallas_call(
        _tcb_body,
        out_shape=jax.ShapeDtypeStruct((VR, 4 * H), jnp.float32),
    )(acc, h1s, dinvb, b1, w2)


NBLK = 8
BLK = VR // NBLK


def _tcc_body(acc_ref, h2s_ref, dinvb_ref, b2_ref, lw1_ref, lb1_ref,
              lw2_ref, lb2_ref, out_ref, pooled_ref):
    i = pl.program_id(0)

    @pl.when(i == 0)
    def _():
        pooled_ref[...] = jnp.zeros((1, 4 * H), jnp.float32)

    pre = (dinvb_ref[...] * (acc_ref[0] + acc_ref[1] + h2s_ref[...])
           + _tile4(b2_ref[...])[None, :])
    act = jnp.tanh(pre)
    g3 = lax.dot_general(
        act, _blockdiag(lw1_ref[...]),
        dimension_numbers=(((1,), (1,)), ((), ())),
        preferred_element_type=jnp.float32,
    )
    h3 = jnp.tanh(g3 + _tile4(lb1_ref[...])[None, :])
    vr = lax.broadcasted_iota(jnp.int32, (BLK, 4 * H), 0) + i * BLK
    h3 = jnp.where(vr < VRN, h3, 0.0)
    pooled_ref[...] += jnp.sum(h3, axis=0, keepdims=True)

    @pl.when(i == NBLK - 1)
    def _():
        lw2t = jnp.concatenate([lw2_ref[...]] * 4, axis=1)
        out_ref[...] = (
            jnp.sum(pooled_ref[...] * lw2t, axis=1, keepdims=True)
            + lb2_ref[...][None, :]
        )


@jax.jit
def _tc_c(acc, h2s, dinvb, b2, lw1, lb1, lw2, lb2):
    row_spec = pl.BlockSpec((BLK, 4 * H), lambda i: (i, 0))
    acc_spec = pl.BlockSpec((2, BLK, 4 * H), lambda i: (0, i, 0))
    small1 = pl.BlockSpec((H,), lambda i: (0,))
    return pl.pallas_call(
        _tcc_body,
        grid=(NBLK,),
        in_specs=[
            acc_spec, row_spec, row_spec,
            small1, pl.BlockSpec((H, H), lambda i: (0, 0)), small1,
            pl.BlockSpec((1, H), lambda i: (0, 0)),
            pl.BlockSpec((1,), lambda i: (0,)),
        ],
        out_specs=pl.BlockSpec((1, 1), lambda i: (0, 0)),
        out_shape=jax.ShapeDtypeStruct((1, 1), jnp.float32),
        scratch_shapes=[pltpu.VMEM((1, 4 * H), jnp.float32)],
    )(acc, h2s, dinvb, b2, lw1, lb1, lw2, lb2)


def kernel(x, edge_index, W1, b1, W2, b2, LW1, Lb1, LW2, Lb2):
    e2d = edge_index.astype(jnp.int32).reshape(2, CROWS, CH)

    degb = _sc_deg(e2d)
    xv = x.reshape(VRN, 4 * D)
    dinvbv, h1sv = _tc_a(degb.reshape(NC, VR, 4 * H), xv, W1)
    acc1 = _sc_msg(e2d, h1sv.reshape(NP, H))
    h2sv = _tc_b(acc1.reshape(NC, VR, 4 * H), h1sv, dinvbv, b1, W2)
    acc2 = _sc_msg(e2d, h2sv.reshape(NP, H))
    out = _tc_c(acc2.reshape(NC, VR, 4 * H), h2sv, dinvbv, b2, LW1, Lb1,
                LW2, Lb2)
    return out.reshape(1)


# speedup vs baseline: 87.9043x; 1.0930x over previous
import jax
import jax.numpy as jnp
from jax import lax
from jax.experimental import pallas as pl
from jax.experimental.pallas import tpu as pltpu
from jax.experimental.pallas import tpu_sc as plsc

N = 10000
D = 128
E = 320000
H = 32

NC = 2
NS = 16
NW = NC * NS

NP = 10240
CH = 128
CROWS = E // CH
RB = CROWS // NW
REXTRA = CROWS - RB * NW
RMAX = RB + 1
NBUF = 4
NRND = RB // NBUF
ROWS_PER_TILE = NP // NS


def _sc_mesh():
    return plsc.VectorSubcoreMesh(core_axis_name="c", subcore_axis_name="s")


CR = 6
CE = CR * CH
NCHK = RB // CR
PW = RB * CH


def _worker_rows(wid):
    base_row = wid * RB + jnp.minimum(wid, REXTRA)
    nrows = jnp.where(wid < REXTRA, RB + 1, RB)
    return base_row, nrows


def _preload_flat(e_flat, plane, base_e, dst_v):
    pltpu.sync_copy(e_flat.at[plane, pl.ds(base_e, PW)], dst_v.at[pl.ds(0, PW)])


def _preload_chunks(e_flat, plane, base_e, dst_v, wid, sem):
    for c in range(NCHK):
        pltpu.async_copy(e_flat.at[plane, pl.ds(base_e + c * CE, CE)],
                         dst_v.at[c], sem)
    for c in range(NCHK):
        pltpu.make_async_copy(e_flat.at[plane, pl.ds(base_e + c * CE, CE)],
                              dst_v.at[c], sem).wait()


def _preload_extra_flat(e_flat, plane, base_e, dst_v, off, wid):
    @pl.when(wid < REXTRA)
    def _():
        pltpu.sync_copy(e_flat.at[plane, pl.ds(base_e + PW, CH)],
                        dst_v.at[pl.ds(off, CH)])


def _deg_body(e_flat, degb_hbm, didx, ones_v, zv, deg_v, degb_v, deg_sh, sem):
    cid = lax.axis_index("c")
    sid = lax.axis_index("s")
    wid = cid * NS + sid
    base_row, nrows = _worker_rows(wid)
    base_e = base_row * CH

    _preload_chunks(e_flat, 1, base_e, didx, wid, sem)

    @pl.when(wid < REXTRA)
    def _():
        pltpu.sync_copy(e_flat.at[1, pl.ds(base_e + PW, CH)],
                        didx.at[NCHK, pl.ds(0, CH)])

    for k in range(CE // 16):
        ones_v[pl.ds(16 * k, 16)] = jnp.full((16,), 1.0, jnp.float32)
    for k in range(CH // 16):
        zv[pl.ds(16 * k, 16)] = jnp.zeros((16,), jnp.float32)
    for i in range(ROWS_PER_TILE // CH):
        pltpu.sync_copy(zv, deg_sh.at[pl.ds(sid * ROWS_PER_TILE + i * CH, CH)])
    plsc.subcore_barrier()

    for c in range(NCHK):
        pltpu.async_copy(ones_v, deg_sh.at[didx.at[c]], sem, add=True)
    for c in range(NCHK):
        pltpu.make_async_copy(ones_v, deg_sh.at[didx.at[c]], sem).wait()

    @pl.when(nrows == RMAX)
    def _():
        pltpu.sync_copy(ones_v.at[pl.ds(0, CH)],
                        deg_sh.at[didx.at[NCHK, pl.ds(0, CH)]], add=True)

    plsc.subcore_barrier()
    pltpu.sync_copy(
        deg_sh.at[pl.ds(sid * ROWS_PER_TILE, ROWS_PER_TILE)], deg_v
    )

    def brow(r, carry):
        row = plsc.load_gather(deg_v, [jnp.full((16,), r, jnp.int32)])
        for k in range(H // 16):
            degb_v[r, pl.ds(16 * k, 16)] = row
        return carry

    lax.fori_loop(0, ROWS_PER_TILE, brow, 0)
    pltpu.sync_copy(
        degb_v, degb_hbm.at[cid, pl.ds(sid * ROWS_PER_TILE, ROWS_PER_TILE)]
    )


@jax.jit
def _sc_deg(e_flat):
    return pl.kernel(
        _deg_body,
        out_type=jax.ShapeDtypeStruct((NC, NP, H), jnp.float32),
        mesh=_sc_mesh(),
        compiler_params=pltpu.CompilerParams(
            use_tc_tiling_on_sc=False, needs_layout_passes=False),
        scratch_types=[
            pltpu.VMEM((NCHK + 1, CE), jnp.int32),
            pltpu.VMEM((CE,), jnp.float32),
            pltpu.VMEM((CH,), jnp.float32),
            pltpu.VMEM((ROWS_PER_TILE,), jnp.float32),
            pltpu.VMEM((ROWS_PER_TILE, H), jnp.float32),
            pltpu.VMEM_SHARED((NP,), jnp.float32),
            pltpu.SemaphoreType.DMA,
        ],
    )(e_flat)


def _msg_body(e_flat, h_hbm, acc_hbm, sidx, didx, rows_v, acc_sh,
              sem_p, sem_g, sem_s):
    cid = lax.axis_index("c")
    sid = lax.axis_index("s")
    wid = cid * NS + sid
    base_row, nrows = _worker_rows(wid)
    base_e = base_row * CH

    _preload_flat(e_flat, 0, base_e, sidx)
    _preload_chunks(e_flat, 1, base_e, didx, wid, sem_p)
    _preload_extra_flat(e_flat, 0, base_e, sidx, PW, wid)

    @pl.when(wid < REXTRA)
    def _():
        pltpu.sync_copy(e_flat.at[1, pl.ds(base_e + PW, CH)],
                        didx.at[NCHK, pl.ds(0, CH)])

    def zrow(i, carry):
        rows_v[0, i, pl.ds(0, 16)] = jnp.zeros((16,), jnp.float32)
        rows_v[0, i, pl.ds(16, 16)] = jnp.zeros((16,), jnp.float32)
        return carry

    lax.fori_loop(0, CH, zrow, 0)
    for i in range(ROWS_PER_TILE // CH):
        pltpu.sync_copy(
            rows_v.at[0, pl.ds(0, CH)],
            acc_sh.at[pl.ds(sid * ROWS_PER_TILE + i * CH, CH)],
        )
    plsc.subcore_barrier()

    def g_start(c, b):
        pltpu.async_copy(h_hbm.at[sidx.at[pl.ds(c * CE, CE)]],
                         rows_v.at[b], sem_g.at[b])

    def g_wait(c, b):
        pltpu.make_async_copy(h_hbm.at[sidx.at[pl.ds(c * CE, CE)]],
                              rows_v.at[b], sem_g.at[b]).wait()

    def s_start(c, b):
        pltpu.async_copy(rows_v.at[b], acc_sh.at[didx.at[c]],
                         sem_s.at[b], add=True)

    def s_wait(c, b):
        pltpu.make_async_copy(rows_v.at[b], acc_sh.at[didx.at[c]],
                              sem_s.at[b]).wait()

    g_start(0, 0)
    for c in range(NCHK):
        b = c % 2
        bn = (c + 1) % 2
        if c >= 1:
            s_wait(c - 1, bn)
        if c + 1 < NCHK:
            g_start(c + 1, bn)
        g_wait(c, b)
        s_start(c, b)

    @pl.when(nrows == RMAX)
    def _():
        bx = NCHK % 2
        pltpu.sync_copy(h_hbm.at[sidx.at[pl.ds(PW, CH)]],
                        rows_v.at[bx, pl.ds(0, CH)])
        pltpu.sync_copy(rows_v.at[bx, pl.ds(0, CH)],
                        acc_sh.at[didx.at[NCHK, pl.ds(0, CH)]], add=True)

    s_wait(NCHK - 1, (NCHK - 1) % 2)

    plsc.subcore_barrier()
    pltpu.sync_copy(
        acc_sh.at[pl.ds(sid * ROWS_PER_TILE, ROWS_PER_TILE)],
        acc_hbm.at[cid, pl.ds(sid * ROWS_PER_TILE, ROWS_PER_TILE)],
    )


@jax.jit
def _sc_msg(e_flat, h):
    return pl.kernel(
        _msg_body,
        out_type=jax.ShapeDtypeStruct((NC, NP, H), jnp.float32),
        mesh=_sc_mesh(),
        compiler_params=pltpu.CompilerParams(use_tc_tiling_on_sc=False),
        scratch_types=[
            pltpu.VMEM((PW + CH,), jnp.int32),
            pltpu.VMEM((NCHK + 1, CE), jnp.int32),
            pltpu.VMEM((2, CE, H), jnp.float32),
            pltpu.VMEM_SHARED((NP, H), jnp.float32),
            pltpu.SemaphoreType.DMA,
            pltpu.SemaphoreType.DMA((2,)),
            pltpu.SemaphoreType.DMA((2,)),
        ],
    )(e_flat, h)


VR = NP // 4
VRN = N // 4


def _blockdiag(w):
    t1 = jnp.concatenate([w, w, w, w], axis=0)
    t2 = jnp.concatenate([t1, t1, t1, t1], axis=1)
    ri = lax.broadcasted_iota(jnp.int32, (4 * H, 4 * H), 0)
    ci = lax.broadcasted_iota(jnp.int32, (4 * H, 4 * H), 1)
    return jnp.where((ri // H) == (ci // H), t2, 0.0)


def _tile4(b):
    return jnp.concatenate([b, b, b, b], axis=0)


def _tca_body(degb_ref, xv_ref, w1_ref, dinvbv_ref, h1sv_ref):
    dinvbv = lax.rsqrt(degb_ref[0] + degb_ref[1] + 1.0)
    t1 = jnp.concatenate([w1_ref[...]] * 4, axis=0)
    t2 = jnp.concatenate([t1] * 4, axis=1)
    ri = lax.broadcasted_iota(jnp.int32, (4 * H, 4 * D), 0)
    ci = lax.broadcasted_iota(jnp.int32, (4 * H, 4 * D), 1)
    w1bd = jnp.where((ri // H) == (ci // D), t2, 0.0)
    g1v = lax.dot_general(
        xv_ref[...], w1bd,
        dimension_numbers=(((1,), (1,)), ((), ())),
        preferred_element_type=jnp.float32,
    )
    g1vf = jnp.concatenate(
        [g1v, jnp.zeros((VR - VRN, 4 * H), jnp.float32)], axis=0)
    dinvbv_ref[...] = dinvbv
    h1sv_ref[...] = g1vf * dinvbv


@jax.jit
def _tc_a(degb, xv, w1):
    return pl.pallas_call(
        _tca_body,
        out_shape=(
            jax.ShapeDtypeStruct((VR, 4 * H), jnp.float32),
            jax.ShapeDtypeStruct((VR, 4 * H), jnp.float32),
        ),
    )(degb, xv, w1)


def _tcb_body(acc_ref, h1s_ref, dinvb_ref, b1_ref, w2_ref, h2s_ref):
    dinvb = dinvb_ref[...]
    pre = (dinvb * (acc_ref[0] + acc_ref[1] + h1s_ref[...])
           + _tile4(b1_ref[...])[None, :])
    act = jnp.tanh(pre)
    g2 = lax.dot_general(
        act, _blockdiag(w2_ref[...]),
        dimension_numbers=(((1,), (1,)), ((), ())),
        preferred_element_type=jnp.float32,
    )
    h2s_ref[...] = g2 * dinvb


@jax.jit
def _tc_b(acc, h1s, dinvb, b1, w2):
    return pl.pallas_call(
        _tcb_body,
        out_shape=jax.ShapeDtypeStruct((VR, 4 * H), jnp.float32),
    )(acc, h1s, dinvb, b1, w2)


def _tcc_body(acc_ref, h2s_ref, dinvb_ref, b2_ref, lw1_ref, lb1_ref,
              lw2_ref, lb2_ref, out_ref):
    pre = (dinvb_ref[...] * (acc_ref[0] + acc_ref[1] + h2s_ref[...])
           + _tile4(b2_ref[...])[None, :])
    act = jnp.tanh(pre)
    g3 = lax.dot_general(
        act, _blockdiag(lw1_ref[...]),
        dimension_numbers=(((1,), (1,)), ((), ())),
        preferred_element_type=jnp.float32,
    )
    h3 = jnp.tanh(g3 + _tile4(lb1_ref[...])[None, :])
    vr = lax.broadcasted_iota(jnp.int32, (VR, 4 * H), 0)
    h3 = jnp.where(vr < VRN, h3, 0.0)
    pooled = jnp.sum(h3, axis=0, keepdims=True)
    lw2t = jnp.concatenate([lw2_ref[...]] * 4, axis=1)
    out_ref[...] = (
        jnp.sum(pooled * lw2t, axis=1, keepdims=True) + lb2_ref[...][None, :]
    )


@jax.jit
def _tc_c(acc, h2s, dinvb, b2, lw1, lb1, lw2, lb2):
    return pl.pallas_call(
        _tcc_body,
        out_shape=jax.ShapeDtypeStruct((1, 1), jnp.float32),
    )(acc, h2s, dinvb, b2, lw1, lb1, lw2, lb2)


def kernel(x, edge_index, W1, b1, W2, b2, LW1, Lb1, LW2, Lb2):
    e_flat = edge_index.astype(jnp.int32)

    degb = _sc_deg(e_flat)
    xv = x.reshape(VRN, 4 * D)
    dinvbv, h1sv = _tc_a(degb.reshape(NC, VR, 4 * H), xv, W1)
    acc1 = _sc_msg(e_flat, h1sv.reshape(NP, H))
    h2sv = _tc_b(acc1.reshape(NC, VR, 4 * H), h1sv, dinvbv, b1, W2)
    acc2 = _sc_msg(e_flat, h2sv.reshape(NP, H))
    out = _tc_c(acc2.reshape(NC, VR, 4 * H), h2sv, dinvbv, b2, LW1, Lb1,
                LW2, Lb2)
    return out.reshape(1)
